# baseline (speedup 1.0000x reference)
import functools

import jax
import jax.numpy as jnp
from jax import lax
from jax.experimental import pallas as pl
from jax.experimental.pallas import tpu as pltpu
from jax.experimental.pallas import tpu_sc as plsc

_SC_CORES = 2
_SC_SUBCORES = 16
_SC_WORKERS = _SC_CORES * _SC_SUBCORES
_LANES = 16
_SC_BLOCK_ROWS = 32


def _sc_partial_sums(pred2, gt2, start_row, n_rows):
    rows, C = pred2.shape
    groups = C // _LANES
    off_blocks = start_row // _SC_BLOCK_ROWS
    mesh = plsc.VectorSubcoreMesh(core_axis_name="c", subcore_axis_name="s")

    @functools.partial(
        pl.kernel,
        mesh=mesh,
        out_type=jax.ShapeDtypeStruct((_SC_WORKERS, _LANES), jnp.float32),
        scratch_types=[pltpu.VMEM((_LANES,), jnp.float32)],
    )
    def k(pred_hbm, gt_hbm, out_hbm, acc_ref):
        wid = lax.axis_index("c") * _SC_SUBCORES + lax.axis_index("s")
        acc_ref[...] = jnp.zeros((_LANES,), jnp.float32)

        def body(pred_v, gt_v):
            d = pred_v[0, pl.ds(0, _LANES)] - gt_v[0, pl.ds(0, _LANES)]
            acc_ref[...] = acc_ref[...] + d * d

        pltpu.emit_pipeline(
            body,
            grid=(n_rows // _SC_BLOCK_ROWS,),
            in_specs=[
                pl.BlockSpec((_SC_BLOCK_ROWS, C), lambda i: (i + off_blocks, 0)),
                pl.BlockSpec((_SC_BLOCK_ROWS, C), lambda i: (i + off_blocks, 0)),
            ],
            out_specs=[],
            core_axis_name=("c", "s"),
            dimension_semantics=(pltpu.PARALLEL,),
        )(pred_hbm, gt_hbm)

        pltpu.sync_copy(acc_ref, out_hbm.at[wid])

    return k(pred2, gt2)


def _tc_loss_kernel(pred_ref, gt_ref, out_ref, acc_ref):
    i = pl.program_id(0)

    @pl.when(i == 0)
    def _():
        acc_ref[0] = 0.0

    g = gt_ref[...]
    d = pred_ref[...] - g
    mask = (g[:, 0:1] != -1.0).astype(jnp.float32)
    acc_ref[0] += jnp.sum(d * d * mask)

    @pl.when(i == pl.num_programs(0) - 1)
    def _():
        out_ref[0, 0] = acc_ref[0]


def _tc_partial(pred2, gt2, start_row, n_rows, block_rows):
    rows, C = pred2.shape
    grid = n_rows // block_rows
    off = start_row // block_rows
    out = pl.pallas_call(
        _tc_loss_kernel,
        grid=(grid,),
        in_specs=[
            pl.BlockSpec((block_rows, C), lambda i: (i + off, 0)),
            pl.BlockSpec((block_rows, C), lambda i: (i + off, 0)),
        ],
        out_specs=pl.BlockSpec((1, 1), lambda i: (0, 0), memory_space=pltpu.SMEM),
        out_shape=jax.ShapeDtypeStruct((1, 1), jnp.float32),
        scratch_shapes=[pltpu.SMEM((1,), jnp.float32)],
    )(pred2, gt2)
    return out[0, 0]


_SC_ROWS = 12288
_TC_BLOCK_ROWS = 1024


def kernel(pred, gt):
    B, N, C = pred.shape
    rows = B * N
    pred2 = pred.reshape(rows, C)
    gt2 = gt.reshape(rows, C)
    tc_rows = rows - _SC_ROWS
    sc_part = _sc_partial_sums(pred2, gt2, 0, rows)
    return jnp.sum(sc_part)


# speedup vs baseline: 1.0061x; 1.0061x over previous
import functools

import jax
import jax.numpy as jnp
from jax import lax
from jax.experimental import pallas as pl
from jax.experimental.pallas import tpu as pltpu
from jax.experimental.pallas import tpu_sc as plsc

_SC_CORES = 2
_SC_SUBCORES = 16
_SC_WORKERS = _SC_CORES * _SC_SUBCORES
_LANES = 16
_RB = 16
_NBUF = 4


def _sc_partial_sums(pred2, gt2, start_row, n_rows):
    rows, C = pred2.shape
    groups = C // _LANES
    rpw = n_rows // _SC_WORKERS
    nsteps = rpw // _RB
    mesh = plsc.VectorSubcoreMesh(core_axis_name="c", subcore_axis_name="s")

    @functools.partial(
        pl.kernel,
        mesh=mesh,
        out_type=jax.ShapeDtypeStruct((_SC_WORKERS, _LANES), jnp.float32),
        scratch_types=[
            pltpu.VMEM((_NBUF, _RB, C), jnp.float32),
            pltpu.VMEM((_NBUF, _RB, C), jnp.float32),
            pltpu.VMEM((_LANES,), jnp.float32),
            pltpu.SemaphoreType.DMA((_NBUF,)),
        ],
    )
    def k(pred_hbm, gt_hbm, out_hbm, pbuf, gbuf, acc_ref, sems):
        wid = lax.axis_index("c") * _SC_SUBCORES + lax.axis_index("s")
        base = start_row + wid * rpw
        acc_ref[...] = jnp.zeros((_LANES,), jnp.float32)

        def start_step(step, b):
            r0 = base + step * _RB
            pltpu.async_copy(pred_hbm.at[pl.ds(r0, _RB)], pbuf.at[b], sems.at[b])
            pltpu.async_copy(gt_hbm.at[pl.ds(r0, _RB)], gbuf.at[b], sems.at[b])

        for b in range(_NBUF - 1):
            start_step(b, b)

        @pl.loop(0, nsteps, step=_NBUF)
        def _(step0):
            for b in range(_NBUF):
                step = step0 + b
                pltpu.make_async_copy(
                    pred_hbm.at[pl.ds(0, _RB)], pbuf.at[b], sems.at[b]
                ).wait()
                pltpu.make_async_copy(
                    gt_hbm.at[pl.ds(0, _RB)], gbuf.at[b], sems.at[b]
                ).wait()
                nxt = step + _NBUF - 1

                @pl.when(nxt < nsteps)
                def _():
                    start_step(nxt, (b + _NBUF - 1) % _NBUF)

                @pl.loop(0, _RB)
                def _(r):
                    s = jnp.zeros((_LANES,), jnp.float32)
                    g0 = None
                    for c in range(groups):
                        sl = pl.ds(c * _LANES, _LANES)
                        g = gbuf[b, r, sl]
                        if c == 0:
                            g0 = g[0]
                        d = pbuf[b, r, sl] - g
                        s = s + d * d
                    m = jnp.where(g0 != -1.0, 1.0, 0.0)
                    acc_ref[...] = acc_ref[...] + s * m

        pltpu.sync_copy(acc_ref, out_hbm.at[wid])

    return k(pred2, gt2)


def kernel(pred, gt):
    B, N, C = pred.shape
    rows = B * N
    pred2 = pred.reshape(rows, C)
    gt2 = gt.reshape(rows, C)
    sc_part = _sc_partial_sums(pred2, gt2, 0, rows)
    return jnp.sum(sc_part)


# speedup vs baseline: 1.4563x; 1.4474x over previous
import jax
import jax.numpy as jnp
from jax.experimental import pallas as pl
from jax.experimental.pallas import tpu as pltpu
from jax.experimental.compute_on import compute_on


def _tc_loss_kernel(pred_ref, gt_ref, out_ref, acc_ref):
    i = pl.program_id(0)

    @pl.when(i == 0)
    def _():
        acc_ref[0] = 0.0

    g = gt_ref[...]
    d = pred_ref[...] - g
    mask = (g[:, 0:1] != -1.0).astype(jnp.float32)
    acc_ref[0] += jnp.sum(d * d * mask)

    @pl.when(i == pl.num_programs(0) - 1)
    def _():
        out_ref[0, 0] = acc_ref[0]


def _tc_partial(pred2, gt2, n_rows, block_rows):
    rows, C = pred2.shape
    grid = n_rows // block_rows
    out = pl.pallas_call(
        _tc_loss_kernel,
        grid=(grid,),
        in_specs=[
            pl.BlockSpec((block_rows, C), lambda i: (i, 0)),
            pl.BlockSpec((block_rows, C), lambda i: (i, 0)),
        ],
        out_specs=pl.BlockSpec((1, 1), lambda i: (0, 0), memory_space=pltpu.SMEM),
        out_shape=jax.ShapeDtypeStruct((1, 1), jnp.float32),
        scratch_shapes=[pltpu.SMEM((1,), jnp.float32)],
    )(pred2, gt2)
    return out[0, 0]


_SC_ROWS = 12288
_TC_BLOCK_ROWS = 2048


def kernel(pred, gt):
    B, N, C = pred.shape
    rows = B * N
    pred2 = pred.reshape(rows, C)
    gt2 = gt.reshape(rows, C)
    tc_rows = rows - _SC_ROWS

    @compute_on("tpu_sparsecore")
    @jax.jit
    def sc_tail(p, g):
        d = p - g
        m = (g[:, 0:1] != -1.0).astype(jnp.float32)
        return jnp.sum(d * d * m)

    sc_part = sc_tail(pred2[tc_rows:], gt2[tc_rows:])
    tc_part = _tc_partial(pred2, gt2, tc_rows, _TC_BLOCK_ROWS)
    return tc_part + sc_part
